# Initial kernel scaffold; baseline (speedup 1.0000x reference)
import jax
import jax.numpy as jnp
from jax.experimental import pallas as pl


def kernel(n_nodes, batch_index, pe):
    raise NotImplementedError("write your pallas kernel here")


# speedup vs baseline: 3.6846x; 3.6846x over previous
import functools

import jax
import jax.numpy as jnp
from jax import lax
from jax.experimental import pallas as pl
from jax.experimental.pallas import tpu as pltpu
from jax.experimental.pallas import tpu_sc as plsc

N_BATCHES = 64
LANES = 16


def _sc_embed(batch_index, pe, *, n, v, d, nw, nc):
    span = -(-n // nw)
    span = -(-span // 64) * 64
    blk = 64
    nblk = span // blk

    mesh = plsc.VectorSubcoreMesh(core_axis_name="c", subcore_axis_name="s")

    def body(bi_hbm, pe_hbm, out_hbm, bi_v, starts_v, idx_v, rows_v, gsem):
        wid = lax.axis_index("s") * nc + lax.axis_index("c")
        base = jnp.minimum(wid * span, n - span)

        pltpu.sync_copy(bi_hbm, bi_v)

        iota = lax.iota(jnp.int32, LANES)
        for g in range(N_BATCHES // LANES):
            tgt = iota + (g * LANES)
            lo = jnp.zeros((LANES,), jnp.int32)
            hi = jnp.full((LANES,), n, jnp.int32)
            for _ in range(17):
                mid = (lo + hi) // 2
                val = plsc.load_gather(bi_v, [jnp.minimum(mid, n - 1)])
                go = jnp.logical_and(val < tgt, lo < hi)
                shrink = jnp.logical_and(jnp.logical_not(val < tgt), lo < hi)
                lo = jnp.where(go, mid + 1, lo)
                hi = jnp.where(shrink, mid, hi)
            starts_v[pl.ds(g * LANES, LANES)] = lo

        def one_block(b, _):
            gblk = base + b * blk
            for j in range(blk // LANES):
                off = gblk + j * LANES
                bvec = bi_v[pl.ds(off, LANES)]
                svec = plsc.load_gather(starts_v, [bvec])
                li = jnp.clip(off + iota - svec, 0, v - 1)
                idx_v[pl.ds(j * LANES, LANES)] = li
            pltpu.async_copy(pe_hbm.at[idx_v], rows_v, gsem).wait()
            pltpu.sync_copy(rows_v, out_hbm.at[pl.ds(gblk, blk)])
            return _

        lax.fori_loop(0, nblk, one_block, 0, unroll=False)

    run = pl.kernel(
        body,
        out_type=jax.ShapeDtypeStruct((n, d), jnp.float32),
        mesh=mesh,
        scratch_types=[
            pltpu.VMEM((n,), jnp.int32),
            pltpu.VMEM((N_BATCHES,), jnp.int32),
            pltpu.VMEM((blk,), jnp.int32),
            pltpu.VMEM((blk, d), jnp.float32),
            pltpu.SemaphoreType.DMA,
        ],
        compiler_params=pltpu.CompilerParams(needs_layout_passes=False),
    )
    return run(batch_index, pe)


def kernel(n_nodes, batch_index, pe):
    n = batch_index.shape[0]
    v, d = pe.shape
    info = plsc.get_sparse_core_info()
    nw = info.num_cores * info.num_subcores
    return _sc_embed(batch_index, pe, n=n, v=v, d=d, nw=nw, nc=info.num_cores)


# speedup vs baseline: 4.8721x; 1.3223x over previous
import jax
import jax.numpy as jnp
from jax import lax
from jax.experimental import pallas as pl
from jax.experimental.pallas import tpu as pltpu
from jax.experimental.pallas import tpu_sc as plsc

N_BATCHES = 64
LANES = 16
BLK = 112


def _sc_embed(batch_index, pe, *, n, v, d, nw, nc):
    span = -(-n // nw)
    span = -(-span // BLK) * BLK
    nblk = span // BLK
    assert nblk % 2 == 0 and (n - span) % 8 == 0 and span <= n

    mesh = plsc.VectorSubcoreMesh(core_axis_name="c", subcore_axis_name="s")

    def body(bi_hbm, pe_hbm, out_hbm, bi_v, starts_v, idx0, idx1, rows0,
             rows1, gsem0, gsem1, wsem0, wsem1):
        wid = lax.axis_index("s") * nc + lax.axis_index("c")
        base = jnp.minimum(wid * span, n - span)

        pltpu.sync_copy(bi_hbm, bi_v)

        iota = lax.iota(jnp.int32, LANES)
        for g in range(N_BATCHES // LANES):
            tgt = iota + (g * LANES)
            lo = jnp.zeros((LANES,), jnp.int32)
            hi = jnp.full((LANES,), n, jnp.int32)
            for _ in range(17):
                mid = (lo + hi) // 2
                val = plsc.load_gather(bi_v, [jnp.minimum(mid, n - 1)])
                lt = lo < hi
                go = jnp.logical_and(val < tgt, lt)
                shrink = jnp.logical_and(jnp.logical_not(val < tgt), lt)
                lo = jnp.where(go, mid + 1, lo)
                hi = jnp.where(shrink, mid, hi)
            starts_v[pl.ds(g * LANES, LANES)] = lo

        idx = (idx0, idx1)
        rows = (rows0, rows1)
        gsem = (gsem0, gsem1)
        wsem = (wsem0, wsem1)

        def compute_idx(b, ib):
            gblk = base + b * BLK
            for j in range(BLK // LANES):
                off = gblk + j * LANES
                bvec = bi_v[pl.ds(off, LANES)]
                svec = plsc.load_gather(starts_v, [bvec])
                li = jnp.clip(off + iota - svec, 0, v - 1)
                ib[pl.ds(j * LANES, LANES)] = li

        def gather(p):
            return pltpu.make_async_copy(pe_hbm.at[idx[p]], rows[p], gsem[p])

        def write(b, p):
            gblk = base + b * BLK
            return pltpu.make_async_copy(
                rows[p], out_hbm.at[pl.ds(gblk, BLK)], wsem[p])

        for p in range(2):
            compute_idx(jnp.int32(p), idx[p])
            gather(p).start()

        def step(i, carry):
            for p in range(2):
                b = i * 2 + p
                gather(p).wait()
                write(b, p).start()
                write(b, p).wait()
                nb = b + 2

                @pl.when(nb < nblk)
                def _(p=p, nb=nb):
                    compute_idx(nb, idx[p])
                    gather(p).start()

            return carry

        lax.fori_loop(0, nblk // 2, step, 0, unroll=False)

    run = pl.kernel(
        body,
        out_type=jax.ShapeDtypeStruct((n, d), jnp.float32),
        mesh=mesh,
        scratch_types=[
            pltpu.VMEM((n,), jnp.int32),
            pltpu.VMEM((N_BATCHES,), jnp.int32),
            pltpu.VMEM((BLK,), jnp.int32),
            pltpu.VMEM((BLK,), jnp.int32),
            pltpu.VMEM((BLK, d), jnp.float32),
            pltpu.VMEM((BLK, d), jnp.float32),
            pltpu.SemaphoreType.DMA,
            pltpu.SemaphoreType.DMA,
            pltpu.SemaphoreType.DMA,
            pltpu.SemaphoreType.DMA,
        ],
        compiler_params=pltpu.CompilerParams(needs_layout_passes=False),
    )
    return run(batch_index, pe)


def kernel(n_nodes, batch_index, pe):
    n = batch_index.shape[0]
    v, d = pe.shape
    info = plsc.get_sparse_core_info()
    nw = info.num_cores * info.num_subcores
    return _sc_embed(batch_index, pe, n=n, v=v, d=d, nw=nw, nc=info.num_cores)


# speedup vs baseline: 8.6850x; 1.7826x over previous
import jax
import jax.numpy as jnp
from jax import lax
from jax.experimental import pallas as pl
from jax.experimental.pallas import tpu as pltpu
from jax.experimental.pallas import tpu_sc as plsc

N_BATCHES = 64
LANES = 16
BLK = 112


def _sc_embed(batch_index, pe, *, n, v, d, nw, nc, ns):
    span = -(-n // nw)
    span = -(-span // BLK) * BLK
    nblk = span // BLK
    assert (n - span) % 8 == 0 and span <= n
    cslice = -(-n // ns)
    cbuf = -(-cslice // 8) * 8 + 8
    pe_rows = v // ns

    mesh = plsc.VectorSubcoreMesh(core_axis_name="c", subcore_axis_name="s")

    def body(bi_hbm, pe_hbm, out_hbm, bi_v, cnt_v, cnt_tmp, cnt_all,
             starts_v, idx_v, rows_v, pe_sp, counts_sp, psem, gsem, wsem):
        cid = lax.axis_index("c")
        sid = lax.axis_index("s")
        wid = sid * nc + cid
        base = jnp.minimum(wid * span, n - span)
        iota = lax.iota(jnp.int32, LANES)

        pe_cp = pltpu.make_async_copy(
            pe_hbm.at[pl.ds(sid * pe_rows, pe_rows)],
            pe_sp.at[pl.ds(sid * pe_rows, pe_rows)], psem)
        pe_cp.start()

        pltpu.sync_copy(bi_hbm.at[pl.ds(base, span)], bi_v)
        cstart = sid * cslice
        cal = jnp.minimum(cstart // 8 * 8, n - cbuf)
        clen = jnp.minimum(cstart + cslice, n) - cal
        pltpu.sync_copy(bi_hbm.at[pl.ds(cal, cbuf)], cnt_v)

        s_lo = jnp.full((LANES,), cstart - cal, jnp.int32)
        s_hi = jnp.full((LANES,), clen, jnp.int32)
        for g in range(N_BATCHES // LANES):
            tgt = iota + (g * LANES)
            lo = s_lo
            hi = s_hi
            for _ in range(14):
                mid = (lo + hi) // 2
                val = plsc.load_gather(cnt_v, [jnp.minimum(mid, cbuf - 1)])
                lt = lo < hi
                go = jnp.logical_and(val < tgt, lt)
                shrink = jnp.logical_and(jnp.logical_not(val < tgt), lt)
                lo = jnp.where(go, mid + 1, lo)
                hi = jnp.where(shrink, mid, hi)
            cnt_tmp[pl.ds(g * LANES, LANES)] = lo - s_lo

        pltpu.sync_copy(cnt_tmp, counts_sp.at[pl.ds(sid * N_BATCHES,
                                                    N_BATCHES)])
        pe_cp.wait()
        plsc.subcore_barrier()
        pltpu.sync_copy(counts_sp, cnt_all)
        for g in range(N_BATCHES // LANES):
            acc = jnp.zeros((LANES,), jnp.int32)
            for t in range(ns):
                acc = acc + cnt_all[pl.ds(t * N_BATCHES + g * LANES, LANES)]
            starts_v[pl.ds(g * LANES, LANES)] = acc

        def one_block(b, nfast):
            gblk = base + b * BLK
            head = bi_v[pl.ds(b * BLK, LANES)]
            tail = bi_v[pl.ds(b * BLK + BLK - LANES, LANES)]
            bmin = jnp.min(head)
            bmax = jnp.max(tail)
            s0v = plsc.load_gather(starts_v, [jnp.full((LANES,), bmin)])
            l0 = gblk - jnp.max(s0v)
            fast = jnp.logical_and(bmin == bmax, l0 + BLK <= v)

            @pl.when(fast)
            def _():
                pltpu.make_async_copy(
                    pe_sp.at[pl.ds(l0, BLK)],
                    out_hbm.at[pl.ds(gblk, BLK)], wsem).start()

            @pl.when(jnp.logical_not(fast))
            def _():
                for j in range(BLK // LANES):
                    off = gblk + j * LANES
                    bvec = bi_v[pl.ds(b * BLK + j * LANES, LANES)]
                    svec = plsc.load_gather(starts_v, [bvec])
                    li = jnp.clip(off + iota - svec, 0, v - 1)
                    idx_v[pl.ds(j * LANES, LANES)] = li
                pltpu.async_copy(pe_hbm.at[idx_v], rows_v, gsem).wait()
                pltpu.sync_copy(rows_v, out_hbm.at[pl.ds(gblk, BLK)])

            return nfast + fast.astype(jnp.int32)

        nfast = lax.fori_loop(0, nblk, one_block, jnp.int32(0), unroll=False)

        def drain(_, carry):
            pltpu.make_async_copy(
                pe_hbm.at[pl.ds(0, BLK)],
                out_hbm.at[pl.ds(base, BLK)], wsem).wait()
            return carry

        lax.fori_loop(0, nfast, drain, 0, unroll=False)

    run = pl.kernel(
        body,
        out_type=jax.ShapeDtypeStruct((n, d), jnp.float32),
        mesh=mesh,
        scratch_types=[
            pltpu.VMEM((span,), jnp.int32),
            pltpu.VMEM((cbuf,), jnp.int32),
            pltpu.VMEM((N_BATCHES,), jnp.int32),
            pltpu.VMEM((ns * N_BATCHES,), jnp.int32),
            pltpu.VMEM((N_BATCHES,), jnp.int32),
            pltpu.VMEM((BLK,), jnp.int32),
            pltpu.VMEM((BLK, d), jnp.float32),
            pltpu.VMEM_SHARED((v, d), jnp.float32),
            pltpu.VMEM_SHARED((ns * N_BATCHES,), jnp.int32),
            pltpu.SemaphoreType.DMA,
            pltpu.SemaphoreType.DMA,
            pltpu.SemaphoreType.DMA,
        ],
        compiler_params=pltpu.CompilerParams(needs_layout_passes=False),
    )
    return run(batch_index, pe)


def kernel(n_nodes, batch_index, pe):
    n = batch_index.shape[0]
    v, d = pe.shape
    info = plsc.get_sparse_core_info()
    return _sc_embed(batch_index, pe, n=n, v=v, d=d,
                     nw=info.num_cores * info.num_subcores,
                     nc=info.num_cores, ns=info.num_subcores)
